# baseline (speedup 1.0000x reference)
import jax
import jax.numpy as jnp
from jax.experimental import pallas as pl

NQ = 4
K = 512
D = 32
B = 131072
BLK = 1024


def _rvq_kernel(zeT_ref, cbn_ref, cbT_ref, idxT_ref, qstackT_ref, qsqT_ref,
                qsq2T_ref, rstatesT_ref, loss_ref):
    i = pl.program_id(0)
    zeT = zeT_ref[...]
    rT = zeT
    iota_sub = jax.lax.broadcasted_iota(jnp.int32, (K, BLK), 0)
    iota_nq = jax.lax.broadcasted_iota(jnp.int32, (1, NQ), 1)
    qsumT = jnp.zeros_like(zeT)
    level_sums = jnp.zeros((1, NQ), jnp.float32)
    for level in range(NQ):
        cb = cbn_ref[level]
        cbt = cbT_ref[level]
        cbsq = jnp.sum(cb * cb, axis=1, keepdims=True)
        rsq = jnp.sum(rT * rT, axis=0, keepdims=True)
        scores2 = jax.lax.dot_general(
            (cb * -2.0).astype(jnp.bfloat16), rT.astype(jnp.bfloat16),
            (((1,), (0,)), ((), ())),
            preferred_element_type=jnp.float32)
        dist = (rsq + cbsq) + scores2
        minv = jnp.min(dist, axis=0, keepdims=True)
        idx = jnp.min(jnp.where(dist == minv, iota_sub, K),
                      axis=0, keepdims=True)
        oh = (iota_sub == idx).astype(jnp.bfloat16)
        hi = cbt.astype(jnp.bfloat16)
        rem = cbt - hi.astype(jnp.float32)
        mid = rem.astype(jnp.bfloat16)
        lo = (rem - mid.astype(jnp.float32)).astype(jnp.bfloat16)
        dims = (((1,), (0,)), ((), ()))
        qT = (jax.lax.dot_general(hi, oh, dims,
                                  preferred_element_type=jnp.float32)
              + jax.lax.dot_general(mid, oh, dims,
                                    preferred_element_type=jnp.float32)
              ) + jax.lax.dot_general(lo, oh, dims,
                                      preferred_element_type=jnp.float32)
        idxT_ref[pl.ds(level, 1), :] = idx
        qstackT_ref[level] = qT
        rstatesT_ref[level] = rT
        diff = rT - qT
        s = jnp.sum(diff * diff)
        level_sums = level_sums + jnp.where(iota_nq == level, s, 0.0)
        rT = diff
        qsumT = qsumT + qT
    rstatesT_ref[NQ] = rT
    qsq = zeT + (qsumT - zeT)
    qsqT_ref[...] = qsq
    qsq2T_ref[...] = qsq

    @pl.when(i == 0)
    def _init():
        loss_ref[...] = level_sums

    @pl.when(i != 0)
    def _acc():
        loss_ref[...] = loss_ref[...] + level_sums


def kernel(ze, codebooks):
    zeT = ze.T
    cbT = jnp.transpose(codebooks, (0, 2, 1))
    grid = (B // BLK,)
    idxT, qstackT, qsqT, qsq2T, rstatesT, loss_sums = pl.pallas_call(
        _rvq_kernel,
        grid=grid,
        in_specs=[
            pl.BlockSpec((D, BLK), lambda i: (0, i)),
            pl.BlockSpec((NQ, K, D), lambda i: (0, 0, 0)),
            pl.BlockSpec((NQ, D, K), lambda i: (0, 0, 0)),
        ],
        out_specs=[
            pl.BlockSpec((NQ, BLK), lambda i: (0, i)),
            pl.BlockSpec((NQ, D, BLK), lambda i: (0, 0, i)),
            pl.BlockSpec((D, BLK), lambda i: (0, i)),
            pl.BlockSpec((D, BLK), lambda i: (0, i)),
            pl.BlockSpec((NQ + 1, D, BLK), lambda i: (0, 0, i)),
            pl.BlockSpec((1, NQ), lambda i: (0, 0)),
        ],
        out_shape=[
            jax.ShapeDtypeStruct((NQ, B), jnp.int32),
            jax.ShapeDtypeStruct((NQ, D, B), jnp.float32),
            jax.ShapeDtypeStruct((D, B), jnp.float32),
            jax.ShapeDtypeStruct((D, B), jnp.float32),
            jax.ShapeDtypeStruct((NQ + 1, D, B), jnp.float32),
            jax.ShapeDtypeStruct((1, NQ), jnp.float32),
        ],
    )(zeT, codebooks, cbT)
    indices = idxT.T
    qstack = jnp.transpose(qstackT, (2, 0, 1))
    qsum = qsqT.T
    qsq = qsq2T.T
    rstates = jnp.transpose(rstatesT, (2, 0, 1))
    per_level = loss_sums[0] / jnp.float32(B * D)
    commitment_loss = jnp.mean(per_level)
    codebook_loss = commitment_loss
    quantization_loss = 0.25 * commitment_loss + codebook_loss
    return (indices, qstack, qsum, qsq, rstates, commitment_loss,
            codebook_loss, quantization_loss)


# speedup vs baseline: 1.5630x; 1.5630x over previous
import jax
import jax.numpy as jnp
from jax.experimental import pallas as pl

NQ = 4
K = 512
D = 32
B = 131072
BLK = 2048


def _rvq_kernel(zeT_ref, cbn_ref, cbT_ref, idxT_ref, qstackT_ref, qsqT_ref,
                qsq2T_ref, rstatesT_ref, loss_ref):
    i = pl.program_id(0)
    zeT = zeT_ref[...]
    rT = zeT
    iota_sub = jax.lax.broadcasted_iota(jnp.int32, (K, BLK), 0)
    iota_nq = jax.lax.broadcasted_iota(jnp.int32, (1, NQ), 1)
    qsumT = jnp.zeros_like(zeT)
    level_sums = jnp.zeros((1, NQ), jnp.float32)
    for level in range(NQ):
        cb = cbn_ref[level]
        cbt = cbT_ref[level]
        cbsq = jnp.sum(cb * cb, axis=1, keepdims=True)
        rsq = jnp.sum(rT * rT, axis=0, keepdims=True)
        scores2 = jax.lax.dot_general(
            (cb * -2.0).astype(jnp.bfloat16), rT.astype(jnp.bfloat16),
            (((1,), (0,)), ((), ())),
            preferred_element_type=jnp.float32)
        dist = (rsq + cbsq) + scores2
        minv = jnp.min(dist, axis=0, keepdims=True)
        idx = jnp.min(jnp.where(dist == minv, iota_sub, K),
                      axis=0, keepdims=True)
        oh = (iota_sub == idx).astype(jnp.bfloat16)
        hi = cbt.astype(jnp.bfloat16)
        rem = cbt - hi.astype(jnp.float32)
        mid = rem.astype(jnp.bfloat16)
        lo = (rem - mid.astype(jnp.float32)).astype(jnp.bfloat16)
        splits = jnp.concatenate([hi, mid, lo], axis=0)
        q3 = jax.lax.dot_general(splits, oh, (((1,), (0,)), ((), ())),
                                 preferred_element_type=jnp.float32)
        qT = (q3[0:D] + q3[D:2 * D]) + q3[2 * D:3 * D]
        idxT_ref[pl.ds(level, 1), :] = idx
        qstackT_ref[level] = qT
        rstatesT_ref[level] = rT
        diff = rT - qT
        s = jnp.sum(diff * diff)
        level_sums = level_sums + jnp.where(iota_nq == level, s, 0.0)
        rT = diff
        qsumT = qsumT + qT
    rstatesT_ref[NQ] = rT
    qsq = zeT + (qsumT - zeT)
    qsqT_ref[...] = qsq
    qsq2T_ref[...] = qsq

    @pl.when(i == 0)
    def _init():
        loss_ref[...] = level_sums

    @pl.when(i != 0)
    def _acc():
        loss_ref[...] = loss_ref[...] + level_sums


def kernel(ze, codebooks):
    zeT = ze.T
    cbT = jnp.transpose(codebooks, (0, 2, 1))
    grid = (B // BLK,)
    idxT, qstackT, qsqT, qsq2T, rstatesT, loss_sums = pl.pallas_call(
        _rvq_kernel,
        grid=grid,
        in_specs=[
            pl.BlockSpec((D, BLK), lambda i: (0, i)),
            pl.BlockSpec((NQ, K, D), lambda i: (0, 0, 0)),
            pl.BlockSpec((NQ, D, K), lambda i: (0, 0, 0)),
        ],
        out_specs=[
            pl.BlockSpec((NQ, BLK), lambda i: (0, i)),
            pl.BlockSpec((NQ, D, BLK), lambda i: (0, 0, i)),
            pl.BlockSpec((D, BLK), lambda i: (0, i)),
            pl.BlockSpec((D, BLK), lambda i: (0, i)),
            pl.BlockSpec((NQ + 1, D, BLK), lambda i: (0, 0, i)),
            pl.BlockSpec((1, NQ), lambda i: (0, 0)),
        ],
        out_shape=[
            jax.ShapeDtypeStruct((NQ, B), jnp.int32),
            jax.ShapeDtypeStruct((NQ, D, B), jnp.float32),
            jax.ShapeDtypeStruct((D, B), jnp.float32),
            jax.ShapeDtypeStruct((D, B), jnp.float32),
            jax.ShapeDtypeStruct((NQ + 1, D, B), jnp.float32),
            jax.ShapeDtypeStruct((1, NQ), jnp.float32),
        ],
    )(zeT, codebooks, cbT)
    indices = idxT.T
    qstack = jnp.transpose(qstackT, (2, 0, 1))
    qsum = qsqT.T
    qsq = qsq2T.T
    rstates = jnp.transpose(rstatesT, (2, 0, 1))
    per_level = loss_sums[0] / jnp.float32(B * D)
    commitment_loss = jnp.mean(per_level)
    codebook_loss = commitment_loss
    quantization_loss = 0.25 * commitment_loss + codebook_loss
    return (indices, qstack, qsum, qsq, rstates, commitment_loss,
            codebook_loss, quantization_loss)
